# Initial kernel scaffold; baseline (speedup 1.0000x reference)
import jax
import jax.numpy as jnp
from jax.experimental import pallas as pl


def kernel(perm, adj, features, appd, w_in, b_in, ln_in_g, ln_in_b, ln_out_g, ln_out_b, w_head, b_head):
    raise NotImplementedError("write your pallas kernel here")


# speedup vs baseline: 2.4730x; 2.4730x over previous
import functools

import jax
import jax.numpy as jnp
from jax.experimental import pallas as pl
from jax.experimental.pallas import tpu as pltpu

_LN_EPS = 1e-5


def _fused_kernel(perm_ref, f_ref, a_ref, w_in_ref, b_in_ref,
                  g_in_ref, be_in_ref, g_out_ref, be_out_ref,
                  w_head_ref, b_head_ref, out_ref, z_ref,
                  *, H, N, d_in, d_model):
    f = f_ref[0]
    a = a_ref[0]
    w_top = w_in_ref[0:d_in, :]
    w_bot = w_in_ref[d_in:2 * d_in, :]
    y = (jnp.dot(f, w_top, preferred_element_type=jnp.float32)
         + jnp.dot(a, w_bot, preferred_element_type=jnp.float32)
         + b_in_ref[...])

    mu = jnp.mean(y, axis=-1, keepdims=True)
    var = jnp.mean((y - mu) ** 2, axis=-1, keepdims=True)
    y = (y - mu) * jax.lax.rsqrt(var + _LN_EPS) * g_in_ref[...] + be_in_ref[...]
    y = jnp.maximum(y, 0.0)

    for h in range(H):
        p = perm_ref[0, h]
        sf = jnp.dot(p, y[h * N:(h + 1) * N, :],
                     preferred_element_type=jnp.float32)
        ob = jax.lax.dot_general(p, sf, (((0,), (0,)), ((), ())),
                                 preferred_element_type=jnp.float32)
        z_ref[:, h * d_model:(h + 1) * d_model] = ob

    z = z_ref[...]
    mu = jnp.mean(z, axis=-1, keepdims=True)
    var = jnp.mean((z - mu) ** 2, axis=-1, keepdims=True)
    zn = (z - mu) * jax.lax.rsqrt(var + _LN_EPS) * g_out_ref[...] + be_out_ref[...]
    out_ref[0] = (jnp.dot(zn, w_head_ref[...], preferred_element_type=jnp.float32)
                  + b_head_ref[...])


def kernel(perm, adj, features, appd, w_in, b_in, ln_in_g, ln_in_b,
           ln_out_g, ln_out_b, w_head, b_head):
    del adj
    B, H, N, _ = perm.shape
    d_in = features.shape[-1]
    d_model = w_in.shape[1]
    nclass = w_head.shape[1]

    f = features.reshape(B, H * N, d_in)
    a = appd.reshape(B, H * N, d_in)

    fused = functools.partial(_fused_kernel, H=H, N=N, d_in=d_in,
                              d_model=d_model)
    return pl.pallas_call(
        fused,
        out_shape=jax.ShapeDtypeStruct((B, N, nclass), jnp.float32),
        grid=(B,),
        in_specs=[
            pl.BlockSpec((1, H, N, N), lambda s: (s, 0, 0, 0)),
            pl.BlockSpec((1, H * N, d_in), lambda s: (s, 0, 0)),
            pl.BlockSpec((1, H * N, d_in), lambda s: (s, 0, 0)),
            pl.BlockSpec((2 * d_in, d_model), lambda s: (0, 0)),
            pl.BlockSpec((1, d_model), lambda s: (0, 0)),
            pl.BlockSpec((1, d_model), lambda s: (0, 0)),
            pl.BlockSpec((1, d_model), lambda s: (0, 0)),
            pl.BlockSpec((1, H * d_model), lambda s: (0, 0)),
            pl.BlockSpec((1, H * d_model), lambda s: (0, 0)),
            pl.BlockSpec((H * d_model, nclass), lambda s: (0, 0)),
            pl.BlockSpec((1, nclass), lambda s: (0, 0)),
        ],
        out_specs=pl.BlockSpec((1, N, nclass), lambda s: (s, 0, 0)),
        scratch_shapes=[pltpu.VMEM((N, H * d_model), jnp.float32)],
        compiler_params=pltpu.CompilerParams(
            dimension_semantics=("parallel",)),
    )(perm, f, a, w_in, b_in, ln_in_g, ln_in_b,
      ln_out_g, ln_out_b, w_head, b_head)


# speedup vs baseline: 3.4654x; 1.4013x over previous
import functools

import jax
import jax.numpy as jnp
from jax.experimental import pallas as pl
from jax.experimental.pallas import tpu as pltpu

_LN_EPS = 1e-5


def _fused_kernel(perm_ref, f_ref, a_ref, w_in_ref, b_in_ref,
                  g_in_ref, be_in_ref, g_out_ref, be_out_ref,
                  w_head_ref, b_head_ref, out_ref, z_ref,
                  *, G, H, N, d_in, d_model):
    f = f_ref[...]
    a = a_ref[...]
    w_top = w_in_ref[0:d_in, :]
    w_bot = w_in_ref[d_in:2 * d_in, :]
    y = (jnp.dot(f, w_top, preferred_element_type=jnp.float32)
         + jnp.dot(a, w_bot, preferred_element_type=jnp.float32)
         + b_in_ref[...])

    mu = jnp.mean(y, axis=-1, keepdims=True)
    var = jnp.mean((y - mu) ** 2, axis=-1, keepdims=True)
    y = (y - mu) * jax.lax.rsqrt(var + _LN_EPS) * g_in_ref[...] + be_in_ref[...]
    y = jnp.maximum(y, 0.0)

    for g in range(G):
        for h in range(H):
            i = g * H + h
            p = perm_ref[i]
            sf = jnp.dot(p, y[i * N:(i + 1) * N, :],
                         preferred_element_type=jnp.float32)
            ob = jax.lax.dot_general(p, sf, (((0,), (0,)), ((), ())),
                                     preferred_element_type=jnp.float32)
            z_ref[g * N:(g + 1) * N, h * d_model:(h + 1) * d_model] = ob

    z = z_ref[...]
    mu = jnp.mean(z, axis=-1, keepdims=True)
    var = jnp.mean((z - mu) ** 2, axis=-1, keepdims=True)
    zn = (z - mu) * jax.lax.rsqrt(var + _LN_EPS) * g_out_ref[...] + be_out_ref[...]
    res = (jnp.dot(zn, w_head_ref[...], preferred_element_type=jnp.float32)
           + b_head_ref[...])
    for g in range(G):
        out_ref[g] = res[g * N:(g + 1) * N, :]


def kernel(perm, adj, features, appd, w_in, b_in, ln_in_g, ln_in_b,
           ln_out_g, ln_out_b, w_head, b_head):
    del adj
    B, H, N, _ = perm.shape
    d_in = features.shape[-1]
    d_model = w_in.shape[1]
    nclass = w_head.shape[1]

    G = 4
    nb = B // G

    p2 = perm.reshape(B * H, N, N)
    f2 = features.reshape(B * H * N, d_in)
    a2 = appd.reshape(B * H * N, d_in)

    fused = functools.partial(_fused_kernel, G=G, H=H, N=N, d_in=d_in,
                              d_model=d_model)
    return pl.pallas_call(
        fused,
        out_shape=jax.ShapeDtypeStruct((B, N, nclass), jnp.float32),
        grid=(nb,),
        in_specs=[
            pl.BlockSpec((G * H, N, N), lambda s: (s, 0, 0)),
            pl.BlockSpec((G * H * N, d_in), lambda s: (s, 0)),
            pl.BlockSpec((G * H * N, d_in), lambda s: (s, 0)),
            pl.BlockSpec((2 * d_in, d_model), lambda s: (0, 0)),
            pl.BlockSpec((1, d_model), lambda s: (0, 0)),
            pl.BlockSpec((1, d_model), lambda s: (0, 0)),
            pl.BlockSpec((1, d_model), lambda s: (0, 0)),
            pl.BlockSpec((1, H * d_model), lambda s: (0, 0)),
            pl.BlockSpec((1, H * d_model), lambda s: (0, 0)),
            pl.BlockSpec((H * d_model, nclass), lambda s: (0, 0)),
            pl.BlockSpec((1, nclass), lambda s: (0, 0)),
        ],
        out_specs=pl.BlockSpec((G, N, nclass), lambda s: (s, 0, 0)),
        scratch_shapes=[pltpu.VMEM((G * N, H * d_model), jnp.float32)],
        compiler_params=pltpu.CompilerParams(
            dimension_semantics=("parallel",)),
    )(p2, f2, a2, w_in, b_in, ln_in_g, ln_in_b,
      ln_out_g, ln_out_b, w_head, b_head)


# speedup vs baseline: 3.6277x; 1.0468x over previous
import functools

import jax
import jax.numpy as jnp
from jax.experimental import pallas as pl
from jax.experimental.pallas import tpu as pltpu

_LN_EPS = 1e-5


def _fused_kernel(perm_ref, f_ref, a_ref, w_in_ref, b_in_ref,
                  g_in_ref, be_in_ref, g_out_ref, be_out_ref,
                  w_head_ref, b_head_ref, out_ref, z_ref,
                  *, G, H, N, d_in, d_model):
    f = f_ref[...]
    a = a_ref[...]
    w_top = w_in_ref[0:d_in, :]
    w_bot = w_in_ref[d_in:2 * d_in, :]
    y = (jnp.dot(f, w_top, preferred_element_type=jnp.float32)
         + jnp.dot(a, w_bot, preferred_element_type=jnp.float32)
         + b_in_ref[...])

    mu = jnp.mean(y, axis=-1, keepdims=True)
    var = jnp.mean((y - mu) ** 2, axis=-1, keepdims=True)
    y = (y - mu) * jax.lax.rsqrt(var + _LN_EPS) * g_in_ref[...] + be_in_ref[...]
    y = jnp.maximum(y, 0.0)

    for g in range(G):
        for h in range(H):
            i = g * H + h
            p = perm_ref[i]
            sf = jnp.dot(p, y[i * N:(i + 1) * N, :],
                         preferred_element_type=jnp.float32)
            ob = jax.lax.dot_general(p, sf, (((0,), (0,)), ((), ())),
                                     preferred_element_type=jnp.float32)
            z_ref[g * N:(g + 1) * N, h * d_model:(h + 1) * d_model] = ob

    z = z_ref[...]
    mu = jnp.mean(z, axis=-1, keepdims=True)
    var = jnp.mean((z - mu) ** 2, axis=-1, keepdims=True)
    zn = (z - mu) * jax.lax.rsqrt(var + _LN_EPS) * g_out_ref[...] + be_out_ref[...]
    res = (jnp.dot(zn, w_head_ref[...], preferred_element_type=jnp.float32)
           + b_head_ref[...])
    for g in range(G):
        out_ref[g] = res[g * N:(g + 1) * N, :]


def kernel(perm, adj, features, appd, w_in, b_in, ln_in_g, ln_in_b,
           ln_out_g, ln_out_b, w_head, b_head):
    del adj
    B, H, N, _ = perm.shape
    d_in = features.shape[-1]
    d_model = w_in.shape[1]
    nclass = w_head.shape[1]

    G = 8
    nb = B // G

    p2 = perm.reshape(B * H, N, N)
    f2 = features.reshape(B * H * N, d_in)
    a2 = appd.reshape(B * H * N, d_in)

    fused = functools.partial(_fused_kernel, G=G, H=H, N=N, d_in=d_in,
                              d_model=d_model)
    return pl.pallas_call(
        fused,
        out_shape=jax.ShapeDtypeStruct((B, N, nclass), jnp.float32),
        grid=(nb,),
        in_specs=[
            pl.BlockSpec((G * H, N, N), lambda s: (s, 0, 0)),
            pl.BlockSpec((G * H * N, d_in), lambda s: (s, 0)),
            pl.BlockSpec((G * H * N, d_in), lambda s: (s, 0)),
            pl.BlockSpec((2 * d_in, d_model), lambda s: (0, 0)),
            pl.BlockSpec((1, d_model), lambda s: (0, 0)),
            pl.BlockSpec((1, d_model), lambda s: (0, 0)),
            pl.BlockSpec((1, d_model), lambda s: (0, 0)),
            pl.BlockSpec((1, H * d_model), lambda s: (0, 0)),
            pl.BlockSpec((1, H * d_model), lambda s: (0, 0)),
            pl.BlockSpec((H * d_model, nclass), lambda s: (0, 0)),
            pl.BlockSpec((1, nclass), lambda s: (0, 0)),
        ],
        out_specs=pl.BlockSpec((G, N, nclass), lambda s: (s, 0, 0)),
        scratch_shapes=[pltpu.VMEM((G * N, H * d_model), jnp.float32)],
        compiler_params=pltpu.CompilerParams(
            dimension_semantics=("parallel",)),
    )(p2, f2, a2, w_in, b_in, ln_in_g, ln_in_b,
      ln_out_g, ln_out_b, w_head, b_head)
